# Initial kernel scaffold; baseline (speedup 1.0000x reference)
import jax
import jax.numpy as jnp
from jax.experimental import pallas as pl


def kernel(node_feat, edge_lengths, radial_cutoff_fn, edge_index, memory_coef, prefactor, invr0):
    raise NotImplementedError("write your pallas kernel here")


# speedup vs baseline: 51.3439x; 51.3439x over previous
import functools

import jax
import jax.numpy as jnp
from jax import lax
from jax.experimental import pallas as pl
from jax.experimental.pallas import tpu as pltpu
from jax.experimental.pallas import tpu_sc as plsc

N_NODES = 100000
DIM = 48
N_EDGES = 1600000
CH = 25600
NPAD = 4 * CH
ACC_ROWS = CH + 16
EPAD = 1638400
EDGES_PER_TILE = EPAD // 16
BATCH = 512
N_BATCHES = EDGES_PER_TILE // BATCH
ZROWS = CH // 16
DRAIN_BLK = 160

_mesh = plsc.VectorSubcoreMesh(core_axis_name="c", subcore_axis_name="s")


@functools.partial(
    pl.kernel,
    out_type=jax.ShapeDtypeStruct((NPAD, DIM), jnp.float32),
    mesh=_mesh,
    compiler_params=pltpu.CompilerParams(use_tc_tiling_on_sc=False),
    scratch_types=[
        pltpu.VMEM((4, 128), jnp.int32),
        pltpu.VMEM((4, 128), jnp.int32),
        pltpu.VMEM((4, 128), jnp.int32),
        pltpu.VMEM((BATCH,), jnp.float32),
        pltpu.VMEM((BATCH,), jnp.float32),
        pltpu.VMEM((BATCH, DIM), jnp.float32),
        pltpu.VMEM((DRAIN_BLK, DIM), jnp.float32),
        pltpu.VMEM((DRAIN_BLK, DIM), jnp.float32),
        pltpu.VMEM((DIM,), jnp.float32),
        pltpu.VMEM((DIM,), jnp.float32),
        pltpu.VMEM((16,), jnp.float32),
        pltpu.VMEM_SHARED((ACC_ROWS, DIM), jnp.float32),
        pltpu.SemaphoreType.DMA,
    ],
)
def _sc_interaction(node_hbm, src_hbm, dst_hbm, len_hbm, cut_hbm,
                    invr_hbm, pref_hbm, mc_hbm, zeros_hbm, out_hbm,
                    srcb, dstb, idxb, lenb, cutb, rows, tmp, nbuf,
                    invrb, prefb, mcb, acc, sem):
    cid = lax.axis_index("c")
    sid = lax.axis_index("s")

    pltpu.sync_copy(invr_hbm, invrb)
    pltpu.sync_copy(pref_hbm, prefb)
    pltpu.sync_copy(mc_hbm, mcb)
    iv = [invrb[pl.ds(16 * q, 16)] for q in range(3)]
    pv = [prefb[pl.ds(16 * q, 16)] for q in range(3)]
    mcv = mcb[...]

    tile_e0 = sid * EDGES_PER_TILE
    tile_r0 = sid * (EDGES_PER_TILE // 128)

    for k in range(2):
        base = (cid * 2 + k) * CH

        pltpu.sync_copy(zeros_hbm, acc.at[pl.ds(sid * ZROWS, ZROWS)])
        @pl.when(sid == 0)
        def _():
            pltpu.sync_copy(zeros_hbm.at[pl.ds(0, 16)],
                            acc.at[pl.ds(CH, 16)])
        plsc.subcore_barrier()

        def batch_body(it, _):
            r0 = tile_r0 + it * 4
            e0 = tile_e0 + it * BATCH
            pltpu.sync_copy(src_hbm.at[pl.ds(r0, 4)], srcb)
            pltpu.sync_copy(dst_hbm.at[pl.ds(r0, 4)], dstb)
            pltpu.sync_copy(len_hbm.at[pl.ds(e0, BATCH)], lenb)
            pltpu.sync_copy(cut_hbm.at[pl.ds(e0, BATCH)], cutb)

            for j in range(4):
                for m in range(8):
                    d = dstb[j, pl.ds(16 * m, 16)]
                    lo = d - base
                    ok = (lo >= 0) & (lo < CH)
                    idxb[j, pl.ds(16 * m, 16)] = jnp.where(ok, lo, CH)

            cps = [
                pltpu.async_copy(node_hbm.at[srcb.at[j]],
                                 rows.at[pl.ds(128 * j, 128)], sem)
                for j in range(4)
            ]
            for cp in cps:
                cp.wait()

            def group_body(g, _):
                g16 = pl.multiple_of(g * 16, 16)
                ln16 = lenb[pl.ds(g16, 16)]
                ct16 = cutb[pl.ds(g16, 16)]
                for t in range(16):
                    e = g16 + t
                    nl = jnp.full((16,), -ln16[t], dtype=jnp.float32)
                    cv = jnp.full((16,), ct16[t], dtype=jnp.float32)
                    for q in range(3):
                        sl = pl.ds(16 * q, 16)
                        mq = jnp.exp(nl * iv[q]) * (pv[q] * cv)
                        rows[e, sl] = rows[e, sl] * mq
                return 0

            lax.fori_loop(0, BATCH // 16, group_body, 0)

            for j in range(4):
                pltpu.sync_copy(rows.at[pl.ds(128 * j, 128)],
                                acc.at[idxb.at[j]], add=True)
            return 0

        lax.fori_loop(0, N_BATCHES, batch_body, 0)
        plsc.subcore_barrier()

        for b in range(10):
            lr0 = sid * (CH // 16) + b * DRAIN_BLK
            gr0 = base + lr0
            pltpu.sync_copy(acc.at[pl.ds(lr0, DRAIN_BLK)], tmp)
            pltpu.sync_copy(node_hbm.at[pl.ds(gr0, DRAIN_BLK)], nbuf)

            def drain_body(r, _):
                for q in range(3):
                    sl = pl.ds(16 * q, 16)
                    tmp[r, sl] = nbuf[r, sl] * mcv + tmp[r, sl]
                return 0

            lax.fori_loop(0, DRAIN_BLK, drain_body, 0)
            pltpu.sync_copy(tmp, out_hbm.at[pl.ds(gr0, DRAIN_BLK)])
        plsc.subcore_barrier()


def kernel(node_feat, edge_lengths, radial_cutoff_fn, edge_index,
           memory_coef, prefactor, invr0):
    n, r, l, c = node_feat.shape
    node2 = node_feat.reshape(n, DIM)
    node_pad = jnp.concatenate(
        [node2, jnp.zeros((NPAD - N_NODES, DIM), jnp.float32)], axis=0)
    pad_e = EPAD - N_EDGES
    src_p = jnp.concatenate(
        [edge_index[0], jnp.zeros((pad_e,), jnp.int32)])
    dst_p = jnp.concatenate(
        [edge_index[1], jnp.full((pad_e,), 2**30, jnp.int32)])
    len_p = jnp.concatenate(
        [edge_lengths, jnp.zeros((pad_e,), jnp.float32)])
    cut_p = jnp.concatenate(
        [radial_cutoff_fn, jnp.zeros((pad_e,), jnp.float32)])
    src2 = src_p.reshape(EPAD // 128, 128)
    dst2 = dst_p.reshape(EPAD // 128, 128)
    invr_flat = jnp.broadcast_to(invr0[:, None, :], (r, l, c)).reshape(DIM)
    pref_flat = jnp.broadcast_to(prefactor[:, None, :], (r, l, c)).reshape(DIM)
    mc16 = jnp.full((16,), memory_coef, jnp.float32)
    zrows = jnp.zeros((ZROWS, DIM), jnp.float32)
    out = _sc_interaction(node_pad, src2, dst2, len_p, cut_p,
                          invr_flat, pref_flat, mc16, zrows)
    return out[:N_NODES].reshape(node_feat.shape)


# speedup vs baseline: 208.2243x; 4.0555x over previous
import functools

import jax
import jax.numpy as jnp
from jax import lax
from jax.experimental import pallas as pl
from jax.experimental.pallas import tpu as pltpu
from jax.experimental.pallas import tpu_sc as plsc

N_NODES = 100000
DIM = 48
N_EDGES = 1600000
CH = 25600
NPAD = 4 * CH
ACC_ROWS = CH + 16
EPAD = 1638400
EDGES_PER_TILE = EPAD // 16
IB = 256
NIB = EDGES_PER_TILE // IB
P = 384
CAP = P + IB
ZROWS = CH // 16
DRAIN_BLK = 160

_mesh = plsc.VectorSubcoreMesh(core_axis_name="c", subcore_axis_name="s")


@functools.partial(
    pl.kernel,
    out_type=jax.ShapeDtypeStruct((NPAD, DIM), jnp.float32),
    mesh=_mesh,
    compiler_params=pltpu.CompilerParams(use_tc_tiling_on_sc=False,
                                         needs_layout_passes=False),
    scratch_types=[
        pltpu.VMEM((2, 128), jnp.int32),
        pltpu.VMEM((2, 128), jnp.int32),
        pltpu.VMEM((2, 128), jnp.int32),
        pltpu.VMEM((2, 128), jnp.int32),
        pltpu.VMEM((IB,), jnp.float32),
        pltpu.VMEM((IB,), jnp.float32),
        pltpu.VMEM((IB,), jnp.float32),
        pltpu.VMEM((IB,), jnp.float32),
        pltpu.VMEM((CAP,), jnp.int32),
        pltpu.VMEM((CAP,), jnp.int32),
        pltpu.VMEM((CAP,), jnp.float32),
        pltpu.VMEM((CAP,), jnp.float32),
        pltpu.VMEM((P,), jnp.float32),
        pltpu.VMEM((P,), jnp.float32),
        pltpu.VMEM((3, 128), jnp.int32),
        pltpu.VMEM((3, 128), jnp.int32),
        pltpu.VMEM((P, DIM), jnp.float32),
        pltpu.VMEM((DRAIN_BLK, DIM), jnp.float32),
        pltpu.VMEM((DRAIN_BLK, DIM), jnp.float32),
        pltpu.VMEM((DIM,), jnp.float32),
        pltpu.VMEM((DIM,), jnp.float32),
        pltpu.VMEM((16,), jnp.float32),
        pltpu.VMEM_SHARED((ACC_ROWS, DIM), jnp.float32),
        pltpu.SemaphoreType.DMA,
        pltpu.SemaphoreType.DMA,
        pltpu.SemaphoreType.DMA,
    ],
)
def _sc_interaction(node_hbm, src_hbm, dst_hbm, len_hbm, cut_hbm,
                    invr_hbm, pref_hbm, mc_hbm, zeros_hbm, out_hbm,
                    sIn0, sIn1, dIn0, dIn1, lnIn0, lnIn1, ctIn0, ctIn1,
                    csrc, cidx, clen, ccut, aplen, apcut, srcb, pidx, rows,
                    tmp, nbuf,
                    invrb, prefb, mcb, acc, semi0, semi1, semg):
    cid = lax.axis_index("c")
    sid = lax.axis_index("s")
    sIn = [sIn0, sIn1]
    dIn = [dIn0, dIn1]
    lnIn = [lnIn0, lnIn1]
    ctIn = [ctIn0, ctIn1]
    semi = [semi0, semi1]

    pltpu.sync_copy(invr_hbm, invrb)
    pltpu.sync_copy(pref_hbm, prefb)
    pltpu.sync_copy(mc_hbm, mcb)
    iv = [invrb[pl.ds(16 * q, 16)] for q in range(3)]
    pv = [prefb[pl.ds(16 * q, 16)] for q in range(3)]
    mcv = mcb[...]
    zero16 = jnp.zeros((16,), jnp.int32)
    iota16 = lax.iota(jnp.int32, 16)
    trash16 = jnp.full((16,), CH, jnp.int32)

    for m in range(CAP // 16):
        csrc[pl.ds(16 * m, 16)] = zero16

    tile_e0 = sid * EDGES_PER_TILE
    tile_r0 = sid * (EDGES_PER_TILE // 128)

    def fire_loads(b, i):
        r0 = tile_r0 + i * 2
        e0 = tile_e0 + i * IB
        pltpu.async_copy(src_hbm.at[pl.ds(r0, 2)], sIn[b], semi[b])
        pltpu.async_copy(dst_hbm.at[pl.ds(r0, 2)], dIn[b], semi[b])
        pltpu.async_copy(len_hbm.at[pl.ds(e0, IB)], lnIn[b], semi[b])
        pltpu.async_copy(cut_hbm.at[pl.ds(e0, IB)], ctIn[b], semi[b])

    def wait_loads(b):
        pltpu.make_async_copy(src_hbm.at[pl.ds(0, 2)], sIn[b], semi[b]).wait()
        pltpu.make_async_copy(dst_hbm.at[pl.ds(0, 2)], dIn[b], semi[b]).wait()
        pltpu.make_async_copy(len_hbm.at[pl.ds(0, IB)], lnIn[b], semi[b]).wait()
        pltpu.make_async_copy(cut_hbm.at[pl.ds(0, IB)], ctIn[b], semi[b]).wait()

    def fire_slot():
        for j in range(3):
            for m in range(8):
                sl = pl.ds(128 * j + 16 * m, 16)
                sl16 = pl.ds(16 * m, 16)
                srcb[j, sl16] = csrc[sl]
                pidx[j, sl16] = cidx[sl]
        for m in range(P // 16):
            sl = pl.ds(16 * m, 16)
            aplen[sl] = clen[sl]
            apcut[sl] = ccut[sl]
        for j in range(3):
            pltpu.async_copy(node_hbm.at[srcb.at[j]],
                             rows.at[pl.ds(128 * j, 128)], semg)
        for m in range((CAP - P) // 16):
            hi = pl.ds(P + 16 * m, 16)
            lo = pl.ds(16 * m, 16)
            csrc[lo] = csrc[hi]
            cidx[lo] = cidx[hi]
            clen[lo] = clen[hi]
            ccut[lo] = ccut[hi]

    def complete_slot():
        for j in range(3):
            pltpu.make_async_copy(node_hbm.at[srcb.at[j]],
                                  rows.at[pl.ds(128 * j, 128)],
                                  semg).wait()

        def apply_body(g, _):
            g16 = pl.multiple_of(g * 16, 16)
            ln16 = aplen[pl.ds(g16, 16)]
            ct16 = apcut[pl.ds(g16, 16)]
            for t in range(16):
                e = g16 + t
                nl = jnp.full((16,), -ln16[t], dtype=jnp.float32)
                cv = jnp.full((16,), ct16[t], dtype=jnp.float32)
                for q in range(3):
                    sl = pl.ds(16 * q, 16)
                    rows[e, sl] = rows[e, sl] * (jnp.exp(nl * iv[q])
                                                 * (pv[q] * cv))
            return 0

        lax.fori_loop(0, P // 16, apply_body, 0)
        for j in range(3):
            pltpu.sync_copy(rows.at[pl.ds(128 * j, 128)],
                            acc.at[pidx.at[j]], add=True)

    for k in range(2):
        base = (cid * 2 + k) * CH

        pltpu.sync_copy(zeros_hbm, acc.at[pl.ds(sid * ZROWS, ZROWS)])
        @pl.when(sid == 0)
        def _():
            pltpu.sync_copy(zeros_hbm.at[pl.ds(0, 16)],
                            acc.at[pl.ds(CH, 16)])
        plsc.subcore_barrier()

        fire_loads(0, 0)
        fire_loads(1, 1)

        def outer_body(o, carry):
            off, pend = carry
            for b in range(2):
                i = o * 2 + b
                wait_loads(b)
                for j in range(2):
                    for m in range(8):
                        sl16 = pl.ds(16 * m, 16)
                        sle = pl.ds(128 * j + 16 * m, 16)
                        d = dIn[b][j, sl16]
                        s = sIn[b][j, sl16]
                        lo = d - base
                        ok = (lo >= 0) & (lo < CH)
                        cntv = plsc.all_reduce_population_count(ok)
                        plsc.store_compressed(csrc.at[pl.ds(off, 16)], s,
                                              mask=ok)
                        plsc.store_compressed(cidx.at[pl.ds(off, 16)], lo,
                                              mask=ok)
                        plsc.store_compressed(clen.at[pl.ds(off, 16)],
                                              lnIn[b][sle], mask=ok)
                        plsc.store_compressed(ccut.at[pl.ds(off, 16)],
                                              ctIn[b][sle], mask=ok)
                        off = off + cntv[0]
                do = off >= P
                @pl.when(do & (pend == 1))
                def _():
                    complete_slot()
                @pl.when(do)
                def _():
                    fire_slot()
                pend = jnp.where(do, jnp.int32(1), pend)
                off = jnp.where(do, off - P, off)
                @pl.when(i + 2 < NIB)
                def _():
                    fire_loads(b, i + 2)
            return off, pend

        off_end, pend_end = lax.fori_loop(
            0, NIB // 2, outer_body, (jnp.int32(0), jnp.int32(0)))
        @pl.when(pend_end == 1)
        def _():
            complete_slot()

        offv = jnp.full((16,), off_end, jnp.int32)
        for m in range(P // 16):
            pos = iota16 + (16 * m)
            msk = pos >= offv
            cidx[pl.ds(16 * m, 16)] = jnp.where(msk, trash16,
                                                cidx[pl.ds(16 * m, 16)])
        fire_slot()
        complete_slot()
        plsc.subcore_barrier()

        for b in range(10):
            lr0 = sid * (CH // 16) + b * DRAIN_BLK
            gr0 = base + lr0
            pltpu.sync_copy(acc.at[pl.ds(lr0, DRAIN_BLK)], tmp)
            pltpu.sync_copy(node_hbm.at[pl.ds(gr0, DRAIN_BLK)], nbuf)

            def drain_body(r, _):
                for q in range(3):
                    sl = pl.ds(16 * q, 16)
                    tmp[r, sl] = nbuf[r, sl] * mcv + tmp[r, sl]
                return 0

            lax.fori_loop(0, DRAIN_BLK, drain_body, 0)
            pltpu.sync_copy(tmp, out_hbm.at[pl.ds(gr0, DRAIN_BLK)])
        plsc.subcore_barrier()


def kernel(node_feat, edge_lengths, radial_cutoff_fn, edge_index,
           memory_coef, prefactor, invr0):
    n, r, l, c = node_feat.shape
    node2 = node_feat.reshape(n, DIM)
    node_pad = jnp.concatenate(
        [node2, jnp.zeros((NPAD - N_NODES, DIM), jnp.float32)], axis=0)
    pad_e = EPAD - N_EDGES
    src_p = jnp.concatenate(
        [edge_index[0], jnp.zeros((pad_e,), jnp.int32)])
    dst_p = jnp.concatenate(
        [edge_index[1], jnp.full((pad_e,), 2**30, jnp.int32)])
    len_p = jnp.concatenate(
        [edge_lengths, jnp.zeros((pad_e,), jnp.float32)])
    cut_p = jnp.concatenate(
        [radial_cutoff_fn, jnp.zeros((pad_e,), jnp.float32)])
    src2 = src_p.reshape(EPAD // 128, 128)
    dst2 = dst_p.reshape(EPAD // 128, 128)
    invr_flat = jnp.broadcast_to(invr0[:, None, :], (r, l, c)).reshape(DIM)
    pref_flat = jnp.broadcast_to(prefactor[:, None, :], (r, l, c)).reshape(DIM)
    mc16 = jnp.full((16,), memory_coef, jnp.float32)
    zrows = jnp.zeros((ZROWS, DIM), jnp.float32)
    out = _sc_interaction(node_pad, src2, dst2, len_p, cut_p,
                          invr_flat, pref_flat, mc16, zrows)
    return out[:N_NODES].reshape(node_feat.shape)
